# Initial kernel scaffold; baseline (speedup 1.0000x reference)
import jax
import jax.numpy as jnp
from jax.experimental import pallas as pl


def kernel(x, table):
    raise NotImplementedError("write your pallas kernel here")


# speedup vs baseline: 2.1557x; 2.1557x over previous
import functools

import jax
import jax.numpy as jnp
from jax import lax
from jax.experimental import pallas as pl
from jax.experimental.pallas import tpu as pltpu
from jax.experimental.pallas import tpu_sc as plsc

N = 1_000_000
D = 20
NB = 2_000
NUM_BLOCKS = N // NB
NW = 32
VECS = NB // 16


def _body(x_hbm, table_hbm, out_hbm, x_v, table_v, out_v):
    wid = lax.axis_index("s") * 2 + lax.axis_index("c")
    pltpu.sync_copy(table_hbm, table_v)
    iota = lax.iota(jnp.int32, 16)

    def do_block(t, _):
        b = wid + t * NW
        base = b * NB
        pltpu.sync_copy(x_hbm.at[pl.ds(base, NB)], x_v)

        def do_vec(v, _):
            xv = x_v[pl.ds(v * 16, 16)]
            idx = jnp.minimum(xv, 5)
            idx = idx + jnp.minimum(xv >> 3, 1)
            idx = idx + jnp.minimum(xv >> 4, 1)
            idx = idx + jnp.minimum(xv >> 5, 1)
            idx = idx + jnp.minimum(xv >> 6, 1)
            idx20 = idx * D
            n20 = (iota + v * 16) * D
            for d in range(D):
                col = plsc.load_gather(table_v, [idx20 + d])
                plsc.store_scatter(out_v, [n20 + d], col)
            return 0

        lax.fori_loop(0, VECS, do_vec, 0)
        pltpu.sync_copy(out_v, out_hbm.at[pl.ds(base * D, NB * D)])
        return 0

    nblocks_w = (NUM_BLOCKS - wid + NW - 1) // NW
    lax.fori_loop(0, nblocks_w, do_block, 0)


def kernel(x, table):
    mesh = plsc.VectorSubcoreMesh(core_axis_name="c", subcore_axis_name="s")
    f = functools.partial(
        pl.kernel,
        mesh=mesh,
        compiler_params=pltpu.CompilerParams(needs_layout_passes=False),
        out_type=jax.ShapeDtypeStruct((N * D,), jnp.float32),
        scratch_types=[
            pltpu.VMEM((NB,), jnp.int32),
            pltpu.VMEM((10 * D,), jnp.float32),
            pltpu.VMEM((NB * D,), jnp.float32),
        ],
    )(_body)
    out = f(x, table.reshape(10 * D))
    return out.reshape(N, D)


# speedup vs baseline: 2.5561x; 1.1858x over previous
import functools

import jax
import jax.numpy as jnp
from jax import lax
from jax.experimental import pallas as pl
from jax.experimental.pallas import tpu as pltpu
from jax.experimental.pallas import tpu_sc as plsc

N = 1_000_000
D = 20
NB = 2_000
NUM_BLOCKS = N // NB
NW = 32
VECS = NB // 16
BPT = 16


def _body(x_hbm, table_hbm, out_hbm, xa, xb, table_v, outa, outb,
          sxa, sxb, soa, sob):
    wid = lax.axis_index("s") * 2 + lax.axis_index("c")
    pltpu.sync_copy(table_hbm, table_v)
    iota20 = lax.iota(jnp.int32, 16) * D

    def base(t):
        return jnp.minimum(wid + t * NW, NUM_BLOCKS - 1) * NB

    def start_x(t, xv, sem):
        pltpu.async_copy(x_hbm.at[pl.ds(base(t), NB)], xv, sem)

    def wait_x(t, xv, sem):
        pltpu.make_async_copy(x_hbm.at[pl.ds(base(t), NB)], xv, sem).wait()

    def start_out(t, ov, sem):
        pltpu.async_copy(ov, out_hbm.at[pl.ds(base(t) * D, NB * D)], sem)

    def wait_out(t, ov, sem):
        pltpu.make_async_copy(ov, out_hbm.at[pl.ds(base(t) * D, NB * D)],
                              sem).wait()

    def compute(xv, ov):
        @plsc.parallel_loop(0, VECS, 1, unroll=5)
        def do_vec(v):
            x16 = xv[pl.ds(v * 16, 16)]
            idx = jnp.minimum(x16, 5)
            idx = idx + jnp.minimum(x16 >> 3, 1)
            idx = idx + jnp.minimum(x16 >> 4, 1)
            idx = idx + jnp.minimum(x16 >> 5, 1)
            idx = idx + jnp.minimum(x16 >> 6, 1)
            idx20 = idx * D
            n20 = iota20 + v * (16 * D)
            cols = [plsc.load_gather(table_v, [idx20 + d]) for d in range(D)]
            for d in range(D):
                plsc.store_scatter(ov, [n20 + d], cols[d])

    start_x(0, xa, sxa)
    start_x(1, xb, sxb)

    def do_pair(i, _):
        tA = 2 * i
        tB = 2 * i + 1
        wait_x(tA, xa, sxa)

        @pl.when(i > 0)
        def _():
            wait_out(tA, outa, soa)

        compute(xa, outa)
        start_out(tA, outa, soa)

        @pl.when(tA + 2 < BPT)
        def _():
            start_x(tA + 2, xa, sxa)

        wait_x(tB, xb, sxb)

        @pl.when(i > 0)
        def _():
            wait_out(tB, outb, sob)

        compute(xb, outb)
        start_out(tB, outb, sob)

        @pl.when(tB + 2 < BPT)
        def _():
            start_x(tB + 2, xb, sxb)

        return 0

    lax.fori_loop(0, BPT // 2, do_pair, 0)
    wait_out(BPT - 2, outa, soa)
    wait_out(BPT - 1, outb, sob)


def kernel(x, table):
    mesh = plsc.VectorSubcoreMesh(core_axis_name="c", subcore_axis_name="s")
    f = functools.partial(
        pl.kernel,
        mesh=mesh,
        compiler_params=pltpu.CompilerParams(needs_layout_passes=False),
        out_type=jax.ShapeDtypeStruct((N * D,), jnp.float32),
        scratch_types=[
            pltpu.VMEM((NB,), jnp.int32),
            pltpu.VMEM((NB,), jnp.int32),
            pltpu.VMEM((10 * D,), jnp.float32),
            pltpu.VMEM((NB * D,), jnp.float32),
            pltpu.VMEM((NB * D,), jnp.float32),
            pltpu.SemaphoreType.DMA,
            pltpu.SemaphoreType.DMA,
            pltpu.SemaphoreType.DMA,
            pltpu.SemaphoreType.DMA,
        ],
    )(_body)
    out = f(x, table.reshape(10 * D))
    return out.reshape(N, D)
